# Initial kernel scaffold; baseline (speedup 1.0000x reference)
import jax
import jax.numpy as jnp
from jax.experimental import pallas as pl


def kernel(x, target):
    raise NotImplementedError("write your pallas kernel here")


# speedup vs baseline: 8.3991x; 8.3991x over previous
import math

import jax
import jax.numpy as jnp
from jax.experimental import pallas as pl
from jax.experimental.pallas import tpu as pltpu

_N_CLASSES = 32000
_PAD = 0
_FILL = 0.1 / (_N_CLASSES - 2)
_CONF = 0.9
_C_ROW = (_N_CLASSES - 2) * _FILL * math.log(_FILL) + _CONF * math.log(_CONF)

_N_ROWS = 4096
_BC = 640
_NBJ = _N_CLASSES // _BC


def _loss_body(x_ref, t_ref, o_ref, acc_ref, xt_ref, x0_ref):
    j = pl.program_id(0)

    @pl.when(j == 0)
    def _init():
        acc_ref[...] = jnp.zeros_like(acc_ref)
        xt_ref[...] = jnp.zeros_like(xt_ref)
        x0_ref[...] = x_ref[:, 0:1]

    t = t_ref[...]
    col0 = j * _BC
    blk = x_ref[...]
    rs = None
    xt = None
    for k in range(_BC // 128):
        sub = blk[:, k * 128:(k + 1) * 128]
        cols = col0 + k * 128 + jax.lax.broadcasted_iota(
            jnp.int32, (_N_ROWS, 128), 1)
        hit = jnp.where(cols == t, sub, 0.0)
        rs = sub if rs is None else rs + sub
        xt = hit if xt is None else xt + hit
    acc_ref[...] += rs
    xt_ref[...] += xt

    @pl.when(j == _NBJ - 1)
    def _fin():
        mask = (t != _PAD).astype(jnp.float32)
        cnt = jnp.sum(mask)
        sum_x0 = jnp.sum(mask * x0_ref[...])
        sum_rs = jnp.sum(mask * acc_ref[...])
        sum_xt = jnp.sum(mask * xt_ref[...])
        o_ref[0, 0] = (_C_ROW * cnt + _FILL * sum_x0 - _FILL * sum_rs
                       + (_FILL - _CONF) * sum_xt)


def _loss_call(x, t2, interpret=False):
    return pl.pallas_call(
        _loss_body,
        grid=(_NBJ,),
        in_specs=[
            pl.BlockSpec((_N_ROWS, _BC), lambda j: (0, j)),
            pl.BlockSpec((_N_ROWS, 1), lambda j: (0, 0)),
        ],
        out_specs=pl.BlockSpec(memory_space=pltpu.SMEM),
        out_shape=jax.ShapeDtypeStruct((1, 1), jnp.float32),
        scratch_shapes=[
            pltpu.VMEM((_N_ROWS, 128), jnp.float32),
            pltpu.VMEM((_N_ROWS, 128), jnp.float32),
            pltpu.VMEM((_N_ROWS, 1), jnp.float32),
        ],
        interpret=interpret,
    )(x, t2)


def kernel(x, target):
    t2 = target.reshape(_N_ROWS, 1).astype(jnp.int32)
    out = _loss_call(x, t2)
    return out[0, 0]
